# Initial kernel scaffold; baseline (speedup 1.0000x reference)
import jax
import jax.numpy as jnp
from jax.experimental import pallas as pl


def kernel(x, edge_index, edge_attr, batch, graph_features, params):
    raise NotImplementedError("write your pallas kernel here")


# speedup vs baseline: 3.1438x; 3.1438x over previous
import functools

import jax
import jax.numpy as jnp
from jax import lax
from jax.experimental import pallas as pl
from jax.experimental.pallas import tpu as pltpu
from jax.experimental.pallas import tpu_sc as plsc

N = 10000
E = 320000
H = 128
DIN = 128
DE = 16
NG = 64
NSYS = 10
NUM_LAYERS = 3

NC = 2
NS = 16
NW = NC * NS
CH = 128
NCHUNK = E // CH
RSUB = 624


_HI = jax.lax.Precision.HIGHEST
_DN0 = (((0,), (0,)), ((), ()))


def _colsum(z):
    n = z.shape[0]
    ones = jnp.ones((n, 1), jnp.float32)
    return lax.dot_general(ones, z, _DN0, preferred_element_type=jnp.float32,
                           precision=_HI)


def _rowsum(z):
    ones = jnp.ones((z.shape[1], 1), jnp.float32)
    return jnp.dot(z, ones, preferred_element_type=jnp.float32, precision=_HI)


def _bn_rows(z, g, b):
    m = jnp.mean(z, axis=0, keepdims=True)
    d = z - m
    v = jnp.mean(d * d, axis=0, keepdims=True)
    return d / jnp.sqrt(v + 1e-5) * g + b


def _prep_body(x_ref, w_ref, b_ref, g_ref, bb_ref, out_ref):
    z = jnp.dot(x_ref[...], w_ref[...], preferred_element_type=jnp.float32)
    z = z + b_ref[...]
    out_ref[...] = jnp.maximum(_bn_rows(z, g_ref[...], bb_ref[...]), 0.0)


def _tc_prep(x, in_w, in_b, bn_g, bn_b):
    return pl.pallas_call(
        _prep_body,
        out_shape=jax.ShapeDtypeStruct((N, H), jnp.float32),
    )(x, in_w, in_b.reshape(1, H), bn_g.reshape(1, H), bn_b.reshape(1, H))


EB = 3200


def _edge_body(ea_ref, ew_ref, eb_ref, w0_ref, b0_ref, w1_ref, b1_ref,
               w2_ref, b2_ref, o0_ref, o1_ref, o2_ref):
    ea = jnp.dot(ea_ref[...], ew_ref[...], preferred_element_type=jnp.float32)
    ea = jnp.maximum(ea + eb_ref[...], 0.0)
    o0_ref[...] = jnp.dot(ea, w0_ref[...], preferred_element_type=jnp.float32) + b0_ref[...]
    o1_ref[...] = jnp.dot(ea, w1_ref[...], preferred_element_type=jnp.float32) + b1_ref[...]
    o2_ref[...] = jnp.dot(ea, w2_ref[...], preferred_element_type=jnp.float32) + b2_ref[...]


def _tc_edge(edge_attr, e_w, e_b, elins):
    full = lambda shape: pl.BlockSpec(shape, lambda i: (0, 0))
    return pl.pallas_call(
        _edge_body,
        grid=(E // EB,),
        in_specs=[
            pl.BlockSpec((EB, DE), lambda i: (i, 0)),
            full((DE, H)), full((1, H)),
            full((H, H)), full((1, H)),
            full((H, H)), full((1, H)),
            full((H, H)), full((1, H)),
        ],
        out_specs=[pl.BlockSpec((EB, H), lambda i: (i, 0))] * 3,
        out_shape=[jax.ShapeDtypeStruct((E, H), jnp.float32)] * 3,
    )(edge_attr, e_w, e_b.reshape(1, H),
      elins[0][0], elins[0][1].reshape(1, H),
      elins[1][0], elins[1][1].reshape(1, H),
      elins[2][0], elins[2][1].reshape(1, H))


def _sc_body(h_hbm, eal_hbm, src_hbm, dst_hbm, out_hbm,
             sidx, didx, ea_buf, g_buf, zbuf, aggr, sem_e, sem_g):
    c = lax.axis_index("c")
    s = lax.axis_index("s")
    w = s * NC + c

    zv = jnp.zeros((16,), jnp.float32)

    def zrow(i, carry):
        for j in range(8):
            zbuf[i, pl.ds(j * 16, 16)] = zv
        return carry

    lax.fori_loop(0, CH, zrow, 0)

    for off, ln in ((0, 128), (128, 128), (256, 128), (384, 128), (512, 112)):
        pltpu.sync_copy(zbuf.at[pl.ds(0, ln), :],
                        aggr.at[pl.ds(s * RSUB + off, ln), :])

    @pl.when(s == NS - 1)
    def _zero_tail():
        pltpu.sync_copy(zbuf.at[pl.ds(0, 16), :], aggr.at[pl.ds(NS * RSUB, 16), :])

    plsc.subcore_barrier()

    nchunks_w = (NCHUNK - w + NW - 1) // NW

    def chunk(t, carry):
        base = (w + t * NW) * CH
        pltpu.sync_copy(src_hbm.at[pl.ds(base, CH)], sidx)
        pltpu.sync_copy(dst_hbm.at[pl.ds(base, CH)], didx)
        cp_e = pltpu.async_copy(eal_hbm.at[pl.ds(base, CH), :], ea_buf, sem_e)
        cp_g = pltpu.async_copy(h_hbm.at[sidx], g_buf, sem_g)
        cp_e.wait()
        cp_g.wait()

        def row(i, rc):
            for j in range(8):
                sl = pl.ds(j * 16, 16)
                g_buf[i, sl] = jnp.maximum(g_buf[i, sl] + ea_buf[i, sl], 0.0)
            return rc

        lax.fori_loop(0, CH, row, 0)
        pltpu.sync_copy(g_buf, aggr.at[didx], add=True)
        return carry

    lax.fori_loop(0, nchunks_w, chunk, 0)
    plsc.subcore_barrier()
    pltpu.sync_copy(aggr.at[pl.ds(s * RSUB, RSUB), :],
                    out_hbm.at[c, pl.ds(s * RSUB, RSUB), :])

    @pl.when(s == NS - 1)
    def _out_tail():
        pltpu.sync_copy(aggr.at[pl.ds(NS * RSUB, 16), :],
                        out_hbm.at[c, pl.ds(NS * RSUB, 16), :])


def _sc_aggregate(h, eal, src, dst):
    mesh = plsc.VectorSubcoreMesh(core_axis_name="c", subcore_axis_name="s")
    f = pl.kernel(
        _sc_body,
        out_type=jax.ShapeDtypeStruct((NC, N, H), jnp.float32),
        mesh=mesh,
        scratch_types=[
            pltpu.VMEM((CH,), jnp.int32),
            pltpu.VMEM((CH,), jnp.int32),
            pltpu.VMEM((CH, H), jnp.float32),
            pltpu.VMEM((CH, H), jnp.float32),
            pltpu.VMEM((CH, H), jnp.float32),
            pltpu.VMEM_SHARED((N, H), jnp.float32),
            pltpu.SemaphoreType.DMA,
            pltpu.SemaphoreType.DMA,
        ],
    )
    return f(h, eal, src, dst)


def _block_body(h_ref, p_ref, eps_ref, w1_ref, b1_ref, g_ref, b_ref,
                w2_ref, b2_ref, lg_ref, lb_ref, out_ref):
    h = h_ref[...]
    aggr = p_ref[0] + p_ref[1]
    z = (1.0 + eps_ref[0, 0]) * h + aggr
    z = jnp.dot(z, w1_ref[...], preferred_element_type=jnp.float32) + b1_ref[...]
    z = jnp.maximum(_bn_rows(z, g_ref[...], b_ref[...]), 0.0)
    z = jnp.dot(z, w2_ref[...], preferred_element_type=jnp.float32) + b2_ref[...]
    m = jnp.mean(z, axis=-1, keepdims=True)
    zd = z - m
    v = jnp.mean(zd * zd, axis=-1, keepdims=True)
    z = zd / jnp.sqrt(v + 1e-5) * lg_ref[...] + lb_ref[...]
    out_ref[...] = jnp.maximum(z + h, 0.0)


def _tc_block(h, parts, bp):
    return pl.pallas_call(
        _block_body,
        out_shape=jax.ShapeDtypeStruct((N, H), jnp.float32),
    )(h, parts, bp['eps'].reshape(1, 1),
      bp['w1'], bp['b1'].reshape(1, H), bp['bn_g'].reshape(1, H), bp['bn_b'].reshape(1, H),
      bp['w2'], bp['b2'].reshape(1, H), bp['ln_g'].reshape(1, H), bp['ln_b'].reshape(1, H))


NB = 2000


def _jk_body(h1_ref, h2_ref, h3_ref,
             jw1_ref, jb1_ref, jw2_ref, jb2_ref, xf_ref):
    hs = (h1_ref[...], h2_ref[...], h3_ref[...])
    a = []
    for hl in hs:
        t = jnp.dot(hl, jw1_ref[...], preferred_element_type=jnp.float32) + jb1_ref[...]
        t = jnp.maximum(t, 0.0)
        a.append(jnp.dot(t, jw2_ref[...], preferred_element_type=jnp.float32) + jb2_ref[...])
    amax = jnp.maximum(jnp.maximum(a[0], a[1]), a[2])
    e = [jnp.exp(ai - amax) for ai in a]
    den = e[0] + e[1] + e[2]
    xf_ref[...] = (e[0] * hs[0] + e[1] * hs[1] + e[2] * hs[2]) / den


def _tc_jk(h1, h2, h3, params):
    nblk = pl.BlockSpec((NB, H), lambda i: (i, 0))
    full = lambda shape: pl.BlockSpec(shape, lambda i: (0, 0))
    return pl.pallas_call(
        _jk_body,
        grid=(N // NB,),
        in_specs=[nblk, nblk, nblk,
                  full((H, H // 2)), full((1, H // 2)),
                  full((H // 2, 1)), full((1, 1))],
        out_specs=nblk,
        out_shape=jax.ShapeDtypeStruct((N, H), jnp.float32),
    )(h1, h2, h3,
      params['jk_w1'], params['jk_b1'].reshape(1, H // 2),
      params['jk_w2'], params['jk_b2'].reshape(1, 1))


def _pool_body(xf_ref, batch_ref, ssum_ref, cnt_ref):
    xf = xf_ref[...]
    bt = batch_ref[...]
    gid = lax.broadcasted_iota(jnp.int32, (N, NG), 1)
    onehot = (bt == gid).astype(jnp.float32)
    ssum_ref[...] = lax.dot_general(onehot, xf, _DN0,
                                    preferred_element_type=jnp.float32, precision=_HI)
    cnt_ref[...] = lax.dot_general(onehot, jnp.ones((N, 1), jnp.float32), _DN0,
                                   preferred_element_type=jnp.float32, precision=_HI)


def _tc_pool(xf, batch):
    return pl.pallas_call(
        _pool_body,
        out_shape=[jax.ShapeDtypeStruct((NG, H), jnp.float32),
                   jax.ShapeDtypeStruct((NG, 1), jnp.float32)],
    )(xf, batch.reshape(N, 1))


GB = 8


def _segmax_body(xf_ref, batch_ref, out_ref):
    gbase = pl.program_id(0) * GB
    xf = xf_ref[...]
    bt = batch_ref[...]
    for k in range(GB):
        sel = jnp.where(bt == gbase + k, xf, jnp.float32(-jnp.inf))
        out_ref[k:k + 1, :] = jnp.max(sel, axis=0, keepdims=True)


def _tc_segmax(xf, batch):
    return pl.pallas_call(
        _segmax_body,
        grid=(NG // GB,),
        in_specs=[pl.BlockSpec((N, H), lambda g: (0, 0)),
                  pl.BlockSpec((N, 1), lambda g: (0, 0))],
        out_specs=pl.BlockSpec((GB, H), lambda g: (g, 0)),
        out_shape=jax.ShapeDtypeStruct((NG, H), jnp.float32),
    )(xf, batch.reshape(N, 1))


def _cls_body(ssum_ref, cnt_ref, mx_ref, gfeat_ref,
              gw_ref, gb_ref,
              wma_ref, wmx_ref, wms_ref, wgf_ref, cb1_ref,
              cg_ref, cbb_ref, w2_ref, b2_ref, out_ref):
    ssum = ssum_ref[...]
    cnt = cnt_ref[...]
    mean = ssum / jnp.maximum(cnt, 1.0)
    mx = jnp.where(cnt > 0.0, mx_ref[...], 0.0)
    gf = jnp.dot(gfeat_ref[...], gw_ref[...], preferred_element_type=jnp.float32) + gb_ref[...]
    z = (jnp.dot(mean, wma_ref[...], preferred_element_type=jnp.float32)
         + jnp.dot(mx, wmx_ref[...], preferred_element_type=jnp.float32)
         + jnp.dot(ssum, wms_ref[...], preferred_element_type=jnp.float32)
         + jnp.dot(gf, wgf_ref[...], preferred_element_type=jnp.float32)
         + cb1_ref[...])
    z = jnp.maximum(_bn_rows(z, cg_ref[...], cbb_ref[...]), 0.0)
    out_ref[...] = jnp.dot(z, w2_ref[...], preferred_element_type=jnp.float32) + b2_ref[...]


def _tc_final(h1, h2, h3, batch, graph_features, params):
    xf = _tc_jk(h1, h2, h3, params)
    ssum, cnt = _tc_pool(xf, batch)
    mx = _tc_segmax(xf, batch)
    w1 = params['cls_w1']
    out = pl.pallas_call(
        _cls_body,
        out_shape=jax.ShapeDtypeStruct((NG, 1), jnp.float32),
    )(ssum, cnt, mx, graph_features,
      params['gf_w'], params['gf_b'].reshape(1, H // 4),
      w1[0:H], w1[H:2 * H], w1[2 * H:3 * H], w1[3 * H:],
      params['cls_b1'].reshape(1, H),
      params['cls_bn_g'].reshape(1, H), params['cls_bn_b'].reshape(1, H),
      params['cls_w2'], params['cls_b2'].reshape(1, 1))
    return out.reshape(NG)


def kernel(x, edge_index, edge_attr, batch, graph_features, params):
    src = edge_index[0]
    dst = edge_index[1]
    h = _tc_prep(x, params['in_w'], params['in_b'], params['in_bn_g'], params['in_bn_b'])
    eals = _tc_edge(edge_attr, params['e_w'], params['e_b'],
                    [(bp['elin_w'], bp['elin_b']) for bp in params['blocks']])
    outs = []
    for l in range(NUM_LAYERS):
        parts = _sc_aggregate(h, eals[l], src, dst)
        h = _tc_block(h, parts, params['blocks'][l])
        outs.append(h)
    return _tc_final(outs[0], outs[1], outs[2], batch, graph_features, params)
